# Initial kernel scaffold; baseline (speedup 1.0000x reference)
import jax
import jax.numpy as jnp
from jax.experimental import pallas as pl


def kernel(x, edge_index, perm, W1, b1, a1, W2, b2, a2):
    raise NotImplementedError("write your pallas kernel here")


# speedup vs baseline: 22.2031x; 22.2031x over previous
import functools

import jax
import jax.numpy as jnp
from jax import lax
from jax.experimental import pallas as pl
from jax.experimental.pallas import tpu as pltpu
from jax.experimental.pallas import tpu_sc as plsc

N = 10000
E = 320000
D = 128
NPAD = 10240
NC = 2
NS = 16
NW = NC * NS

BLK = 512
GRID = NPAD // BLK

EPW = E // NW
EPT = E // NS
CH = 80
NCHUNK = EPT // CH
RPT = NPAD // NS
RPW = NPAD // NW


def _mesh():
    return plsc.VectorSubcoreMesh(
        core_axis_name="c", subcore_axis_name="s",
        num_cores=NC, num_subcores=NS)


def _stage_a_body(dst_hbm, perm_hbm, h1_hbm, degp_hbm, h1n_hbm,
                  cnt_v, idx_v, permb, rows_v, sem):
    c = lax.axis_index("c")
    s = lax.axis_index("s")
    w = s * NC + c

    zeros16 = jnp.zeros((16,), jnp.float32)

    def zbody(i, _):
        cnt_v[pl.ds(pl.multiple_of(i * 16, 16), 16)] = zeros16
        return 0
    lax.fori_loop(0, NPAD // 16, zbody, 0)

    pltpu.sync_copy(dst_hbm.at[pl.ds(pl.multiple_of(w * EPW, 8), EPW)], idx_v)

    ones16 = jnp.ones((16,), jnp.float32)

    def cbody(i, _):
        idx = idx_v[pl.ds(pl.multiple_of(i * 16, 16), 16)]
        plsc.addupdate_scatter(cnt_v, [idx], ones16)
        return 0
    lax.fori_loop(0, EPW // 16, cbody, 0)

    pltpu.sync_copy(cnt_v, degp_hbm.at[w])

    for k in range(RPW // CH):
        base = w * RPW + k * CH
        pltpu.sync_copy(perm_hbm.at[pl.ds(pl.multiple_of(base, 8), CH)], permb)
        pltpu.async_copy(h1_hbm.at[permb], rows_v, sem).wait()
        pltpu.sync_copy(rows_v, h1n_hbm.at[pl.ds(pl.multiple_of(base, 8), CH)])


def _stage_a(dst, permp, h1):
    f = pl.kernel(
        _stage_a_body,
        out_type=(jax.ShapeDtypeStruct((NW, NPAD), jnp.float32),
                  jax.ShapeDtypeStruct((NPAD, D), jnp.float32)),
        mesh=_mesh(),
        scratch_types=[
            pltpu.VMEM((NPAD,), jnp.float32),
            pltpu.VMEM((EPW,), jnp.int32),
            pltpu.VMEM((CH,), jnp.int32),
            pltpu.VMEM((CH, D), jnp.float32),
            pltpu.SemaphoreType.DMA,
        ],
        compiler_params=pltpu.CompilerParams(needs_layout_passes=False),
    )
    return f(dst, permp, h1)


SEG = 4000
NSEG = EPT // SEG
SEGP = SEG // (2 * CH)


def _scatter_body(src_hbm, dst_hbm, g_hbm, zrow_hbm, out_hbm,
                  srcall, dstall, dstb0, dstb1,
                  rows0, rows1, acc_sh, sem0, sem1):
    c = lax.axis_index("c")
    s = lax.axis_index("s")
    goff = c * NPAD

    pltpu.sync_copy(zrow_hbm, rows0)
    for k in range(RPT // CH):
        pltpu.sync_copy(
            rows0, acc_sh.at[pl.ds(pl.multiple_of(s * RPT + k * CH, 8), CH)])
    plsc.subcore_barrier()

    def stage_dst(j, db):
        for t in range(CH // 16):
            o = pl.multiple_of(j * CH + t * 16, 16)
            db[pl.ds(t * 16, 16)] = dstall[pl.ds(o, 16)]

    def fire(j, rows, sem):
        pltpu.async_copy(
            g_hbm.at[srcall.at[pl.ds(pl.multiple_of(j * CH, 8), CH)]],
            rows, sem)

    def wait(j, rows, sem):
        pltpu.make_async_copy(
            g_hbm.at[srcall.at[pl.ds(pl.multiple_of(j * CH, 8), CH)]],
            rows, sem).wait()

    def scat(db, rows):
        pltpu.sync_copy(rows, acc_sh.at[db], add=True)

    for seg in range(NSEG):
        ebase = pl.multiple_of(s * EPT + seg * SEG, 8)
        pltpu.sync_copy(src_hbm.at[pl.ds(ebase, SEG)], srcall)
        pltpu.sync_copy(dst_hbm.at[pl.ds(ebase, SEG)], dstall)
        def addoff(i, _):
            o = pl.multiple_of(i * 16, 16)
            srcall[pl.ds(o, 16)] = srcall[pl.ds(o, 16)] + goff
            return 0
        lax.fori_loop(0, SEG // 16, addoff, 0)

        stage_dst(0, dstb0)
        fire(0, rows0, sem0)

        def pair(jj, _):
            j0 = jj * 2
            stage_dst(j0 + 1, dstb1)
            fire(j0 + 1, rows1, sem1)
            wait(j0, rows0, sem0)
            scat(dstb0, rows0)

            @pl.when(jj < SEGP - 1)
            def _():
                stage_dst(j0 + 2, dstb0)
                fire(j0 + 2, rows0, sem0)

            wait(j0 + 1, rows1, sem1)
            scat(dstb1, rows1)
            return 0
        lax.fori_loop(0, SEGP, pair, 0)

    plsc.subcore_barrier()
    for k in range(RPT // CH):
        r0 = pl.multiple_of(s * RPT + k * CH, 8)
        pltpu.sync_copy(acc_sh.at[pl.ds(r0, CH)], rows0)
        pltpu.sync_copy(rows0, out_hbm.at[pl.ds(goff + r0, CH)])


def _scatter(src, dst, gflat, zrow):
    f = pl.kernel(
        _scatter_body,
        out_type=jax.ShapeDtypeStruct((NC * NPAD, D), jnp.float32),
        mesh=_mesh(),
        scratch_types=[
            pltpu.VMEM((SEG,), jnp.int32),
            pltpu.VMEM((SEG,), jnp.int32),
            pltpu.VMEM((CH,), jnp.int32),
            pltpu.VMEM((CH,), jnp.int32),
            pltpu.VMEM((CH, D), jnp.float32),
            pltpu.VMEM((CH, D), jnp.float32),
            pltpu.VMEM_SHARED((NPAD, D), jnp.float32),
            pltpu.SemaphoreType.DMA,
            pltpu.SemaphoreType.DMA,
        ],
        compiler_params=pltpu.CompilerParams(needs_layout_passes=False),
    )
    return f(src, dst, gflat, zrow)


def _mm_body(x_ref, w_ref, o_ref):
    o_ref[...] = jnp.dot(x_ref[...], w_ref[...],
                         preferred_element_type=jnp.float32)


def _mm(xp, W):
    return pl.pallas_call(
        _mm_body,
        grid=(GRID,),
        in_specs=[pl.BlockSpec((BLK, D), lambda i: (i, 0)),
                  pl.BlockSpec((D, D), lambda i: (0, 0))],
        out_specs=pl.BlockSpec((BLK, D), lambda i: (i, 0)),
        out_shape=jax.ShapeDtypeStruct((NPAD, D), jnp.float32),
    )(xp, W)


def _prep1_body(degp_ref, h1_ref, h1n_ref, g_ref, dinv_ref):
    deg = jnp.sum(degp_ref[...], axis=0) + 1.0
    dv = lax.rsqrt(deg)[:, None]
    dinv_ref[...] = dv
    g_ref[0] = h1_ref[...] * dv
    g_ref[1] = h1n_ref[...] * dv


def _prep1(degp, h1, h1n):
    return pl.pallas_call(
        _prep1_body,
        grid=(GRID,),
        in_specs=[pl.BlockSpec((NW, BLK), lambda i: (0, i)),
                  pl.BlockSpec((BLK, D), lambda i: (i, 0)),
                  pl.BlockSpec((BLK, D), lambda i: (i, 0))],
        out_specs=[pl.BlockSpec((2, BLK, D), lambda i: (0, i, 0)),
                   pl.BlockSpec((BLK, 1), lambda i: (i, 0))],
        out_shape=[jax.ShapeDtypeStruct((2, NPAD, D), jnp.float32),
                   jax.ShapeDtypeStruct((NPAD, 1), jnp.float32)],
    )(degp, h1, h1n)


def _prep2_body(acc_ref, g_ref, dinv_ref, w2_ref, b_ref, a_ref, o_ref):
    dv = dinv_ref[...]
    for k in range(2):
        z = dv * (acc_ref[k] + g_ref[k]) + b_ref[...]
        z = jnp.where(z >= 0, z, a_ref[...] * z)
        o_ref[k] = dv * jnp.dot(z, w2_ref[...],
                                preferred_element_type=jnp.float32)


def _prep2(acc, g, dinv, W2, b1, a1):
    return pl.pallas_call(
        _prep2_body,
        grid=(GRID,),
        in_specs=[pl.BlockSpec((2, BLK, D), lambda i: (0, i, 0)),
                  pl.BlockSpec((2, BLK, D), lambda i: (0, i, 0)),
                  pl.BlockSpec((BLK, 1), lambda i: (i, 0)),
                  pl.BlockSpec((D, D), lambda i: (0, 0)),
                  pl.BlockSpec((1, D), lambda i: (0, 0)),
                  pl.BlockSpec((1, D), lambda i: (0, 0))],
        out_specs=pl.BlockSpec((2, BLK, D), lambda i: (0, i, 0)),
        out_shape=jax.ShapeDtypeStruct((2, NPAD, D), jnp.float32),
    )(acc, g, dinv, W2, b1, a1)


def _final_body(acc_ref, g_ref, dinv_ref, b_ref, a_ref,
                pos_ref, neg_ref, sum_ref):
    i = pl.program_id(0)
    dv = dinv_ref[...]
    zp = dv * (acc_ref[0] + g_ref[0]) + b_ref[...]
    zp = jnp.where(zp >= 0, zp, a_ref[...] * zp)
    zn = dv * (acc_ref[1] + g_ref[1]) + b_ref[...]
    zn = jnp.where(zn >= 0, zn, a_ref[...] * zn)
    pos_ref[...] = zp
    neg_ref[...] = zn
    rows = lax.broadcasted_iota(jnp.int32, (BLK, 1), 0) + i * BLK
    part = jnp.sum(jnp.where(rows < N, zp, 0.0), axis=0, keepdims=True)

    @pl.when(i == 0)
    def _():
        sum_ref[...] = jnp.zeros_like(sum_ref)
    sum_ref[...] += part

    @pl.when(i == GRID - 1)
    def _():
        sum_ref[...] = jax.nn.sigmoid(sum_ref[...] * (1.0 / N))


def _final(acc, g, dinv, b2, a2):
    return pl.pallas_call(
        _final_body,
        grid=(GRID,),
        in_specs=[pl.BlockSpec((2, BLK, D), lambda i: (0, i, 0)),
                  pl.BlockSpec((2, BLK, D), lambda i: (0, i, 0)),
                  pl.BlockSpec((BLK, 1), lambda i: (i, 0)),
                  pl.BlockSpec((1, D), lambda i: (0, 0)),
                  pl.BlockSpec((1, D), lambda i: (0, 0))],
        out_specs=[pl.BlockSpec((BLK, D), lambda i: (i, 0)),
                   pl.BlockSpec((BLK, D), lambda i: (i, 0)),
                   pl.BlockSpec((1, D), lambda i: (0, 0))],
        out_shape=[jax.ShapeDtypeStruct((NPAD, D), jnp.float32),
                   jax.ShapeDtypeStruct((NPAD, D), jnp.float32),
                   jax.ShapeDtypeStruct((1, D), jnp.float32)],
    )(acc, g, dinv, b2, a2)


@functools.partial(jax.jit)
def kernel(x, edge_index, perm, W1, b1, a1, W2, b2, a2):
    x = x.astype(jnp.float32)
    src = edge_index[0].astype(jnp.int32)
    dst = edge_index[1].astype(jnp.int32)
    permp = jnp.pad(perm.astype(jnp.int32), (0, NPAD - N))
    xp = jnp.pad(x, ((0, NPAD - N), (0, 0)))
    zrow = jnp.zeros((CH, D), jnp.float32)
    b1r = b1.reshape(1, D)
    a1r = a1.reshape(1, D)
    b2r = b2.reshape(1, D)
    a2r = a2.reshape(1, D)

    h1 = _mm(xp, W1)
    degp, h1n = _stage_a(dst, permp, h1)
    g1, dinv = _prep1(degp, h1, h1n)
    acc1 = _scatter(src, dst, g1.reshape(NC * NPAD, D), zrow)
    g2 = _prep2(acc1.reshape(2, NPAD, D), g1, dinv, W2, b1r, a1r)
    acc2 = _scatter(src, dst, g2.reshape(NC * NPAD, D), zrow)
    posz, negz, summ = _final(acc2.reshape(2, NPAD, D), g2, dinv, b2r, a2r)
    return posz[:N], negz[:N], summ.reshape(D)
